# Initial kernel scaffold; baseline (speedup 1.0000x reference)
import jax
import jax.numpy as jnp
from jax.experimental import pallas as pl


def kernel(x, state, node_emb, gate_w, gate_b, upd_w, upd_b):
    raise NotImplementedError("write your pallas kernel here")


# speedup vs baseline: 2.0353x; 2.0353x over previous
import jax
import jax.numpy as jnp
from jax.experimental import pallas as pl
from jax.experimental.pallas import tpu as pltpu

f32 = jnp.float32


def _prep_a_kernel(e_ref, xf_ref, gb_ref, ub_ref,
                   a_ref, aggx_ref, bg_ref, bu_ref):
    e = e_ref[...]
    g = jax.lax.dot_general(e, e, (((1,), (1,)), ((), ())),
                            preferred_element_type=f32)
    g = jnp.maximum(g, 0.0)
    g = g - jnp.max(g, axis=1, keepdims=True)
    eg = jnp.exp(g)
    a = eg / jnp.sum(eg, axis=1, keepdims=True)
    a_ref[...] = a
    aggx_ref[...] = jnp.dot(a, xf_ref[...], preferred_element_type=f32)
    bg_ref[...] = jnp.dot(e, gb_ref[...], preferred_element_type=f32)
    bu_ref[...] = jnp.dot(e, ub_ref[...], preferred_element_type=f32)


def _weff_kernel(e_ref, gcat_ref, ucat_ref, wg_ref, wu_ref):
    e = e_ref[...]
    wg_ref[...] = jnp.dot(e, gcat_ref[...], preferred_element_type=f32)
    wu_ref[...] = jnp.dot(e, ucat_ref[...], preferred_element_type=f32)


def _agg_kernel(a_ref, v_ref, o_ref):
    o_ref[...] = jnp.dot(a_ref[...], v_ref[...], preferred_element_type=f32)


def _gate_kernel(s_ref, aggs_ref, x_ref, aggx_ref, wg_ref, bg_ref,
                 t_ref, r_ref):
    nb = s_ref.shape[0]
    for j in range(nb):
        s = s_ref[j]
        cat = jnp.concatenate(
            [s, aggs_ref[j], x_ref[j], aggx_ref[j]], axis=0)
        pre = jax.lax.dot_general(wg_ref[j], cat, (((0,), (0,)), ((), ())),
                                  preferred_element_type=f32)
        zr = jax.nn.sigmoid(pre + bg_ref[j])
        h = s.shape[0]
        z = zr[:h, :]
        t_ref[j] = z * s
        r_ref[j] = zr[h:, :]


def _cand_kernel(t_ref, aggt_ref, x_ref, aggx_ref, wu_ref, bu_ref,
                 r_ref, s_ref, h_ref):
    nb = t_ref.shape[0]
    for j in range(nb):
        cat = jnp.concatenate(
            [t_ref[j], aggt_ref[j], x_ref[j], aggx_ref[j]], axis=0)
        pre = jax.lax.dot_general(wu_ref[j], cat, (((0,), (0,)), ((), ())),
                                  preferred_element_type=f32)
        hc = jnp.tanh(pre + bu_ref[j])
        r = r_ref[j]
        h_ref[j] = r * s_ref[j] + (1.0 - r) * hc


def kernel(x, state, node_emb, gate_w, gate_b, upd_w, upd_b):
    b, n, ci = x.shape
    h = state.shape[-1]
    d = node_emb.shape[-1]
    out_dtype = state.dtype
    kc = 2 * h + 2 * ci

    e = node_emb.astype(f32)
    x_t = x.astype(f32).transpose(1, 2, 0)
    x_flat = x_t.reshape(n, ci * b)
    s_t = state.astype(f32).transpose(1, 2, 0)
    s_flat = s_t.reshape(n, h * b)

    gw = gate_w.astype(f32)
    uw = upd_w.astype(f32)
    gcat = jnp.concatenate(
        [gw[:, 0, ci:, :], gw[:, 1, ci:, :],
         gw[:, 0, :ci, :], gw[:, 1, :ci, :]], axis=1).reshape(d, kc * 2 * h)
    ucat = jnp.concatenate(
        [uw[:, 0, ci:, :], uw[:, 1, ci:, :],
         uw[:, 0, :ci, :], uw[:, 1, :ci, :]], axis=1).reshape(d, kc * h)

    vmem = pl.BlockSpec(memory_space=pltpu.MemorySpace.VMEM)
    a_adj, aggx_flat, bg, bu = pl.pallas_call(
        _prep_a_kernel,
        out_shape=(jax.ShapeDtypeStruct((n, n), f32),
                   jax.ShapeDtypeStruct((n, ci * b), f32),
                   jax.ShapeDtypeStruct((n, 2 * h), f32),
                   jax.ShapeDtypeStruct((n, h), f32)),
        in_specs=[vmem] * 4,
        out_specs=(vmem,) * 4,
    )(e, x_flat, gate_b.astype(f32), upd_b.astype(f32))
    bg3 = bg.reshape(n, 2 * h, 1)
    bu3 = bu.reshape(n, h, 1)

    nb_w = 64 if n % 64 == 0 else n
    wg_flat, wu_flat = pl.pallas_call(
        _weff_kernel,
        grid=(n // nb_w,),
        out_shape=(jax.ShapeDtypeStruct((n, kc * 2 * h), f32),
                   jax.ShapeDtypeStruct((n, kc * h), f32)),
        in_specs=[
            pl.BlockSpec((nb_w, d), lambda i: (i, 0)),
            pl.BlockSpec((d, kc * 2 * h), lambda i: (0, 0)),
            pl.BlockSpec((d, kc * h), lambda i: (0, 0)),
        ],
        out_specs=(pl.BlockSpec((nb_w, kc * 2 * h), lambda i: (i, 0)),
                   pl.BlockSpec((nb_w, kc * h), lambda i: (i, 0))),
        compiler_params=pltpu.CompilerParams(
            dimension_semantics=("parallel",)),
    )(e, gcat, ucat)
    wg3 = wg_flat.reshape(n, kc, 2 * h)
    wu3 = wu_flat.reshape(n, kc, h)

    def agg(v_flat):
        cols = v_flat.shape[1]
        cb = 2048 if cols % 2048 == 0 else cols
        return pl.pallas_call(
            _agg_kernel,
            grid=(cols // cb,),
            out_shape=jax.ShapeDtypeStruct((n, cols), f32),
            in_specs=[
                pl.BlockSpec((n, n), lambda i: (0, 0)),
                pl.BlockSpec((n, cb), lambda i: (0, i)),
            ],
            out_specs=pl.BlockSpec((n, cb), lambda i: (0, i)),
            compiler_params=pltpu.CompilerParams(
                dimension_semantics=("parallel",)),
        )(a_adj, v_flat)

    aggs3 = agg(s_flat).reshape(n, h, b)
    aggx3 = aggx_flat.reshape(n, ci, b)

    nb_g = 32 if n % 32 == 0 else n
    t3, r3 = pl.pallas_call(
        _gate_kernel,
        grid=(n // nb_g,),
        out_shape=(jax.ShapeDtypeStruct((n, h, b), f32),
                   jax.ShapeDtypeStruct((n, h, b), f32)),
        in_specs=[
            pl.BlockSpec((nb_g, h, b), lambda i: (i, 0, 0)),
            pl.BlockSpec((nb_g, h, b), lambda i: (i, 0, 0)),
            pl.BlockSpec((nb_g, ci, b), lambda i: (i, 0, 0)),
            pl.BlockSpec((nb_g, ci, b), lambda i: (i, 0, 0)),
            pl.BlockSpec((nb_g, kc, 2 * h), lambda i: (i, 0, 0)),
            pl.BlockSpec((nb_g, 2 * h, 1), lambda i: (i, 0, 0)),
        ],
        out_specs=(pl.BlockSpec((nb_g, h, b), lambda i: (i, 0, 0)),
                   pl.BlockSpec((nb_g, h, b), lambda i: (i, 0, 0))),
        compiler_params=pltpu.CompilerParams(
            dimension_semantics=("parallel",)),
    )(s_t, aggs3, x_t, aggx3, wg3, bg3)

    aggt3 = agg(t3.reshape(n, h * b)).reshape(n, h, b)

    h3 = pl.pallas_call(
        _cand_kernel,
        grid=(n // nb_g,),
        out_shape=jax.ShapeDtypeStruct((n, h, b), f32),
        in_specs=[
            pl.BlockSpec((nb_g, h, b), lambda i: (i, 0, 0)),
            pl.BlockSpec((nb_g, h, b), lambda i: (i, 0, 0)),
            pl.BlockSpec((nb_g, ci, b), lambda i: (i, 0, 0)),
            pl.BlockSpec((nb_g, ci, b), lambda i: (i, 0, 0)),
            pl.BlockSpec((nb_g, kc, h), lambda i: (i, 0, 0)),
            pl.BlockSpec((nb_g, h, 1), lambda i: (i, 0, 0)),
            pl.BlockSpec((nb_g, h, b), lambda i: (i, 0, 0)),
            pl.BlockSpec((nb_g, h, b), lambda i: (i, 0, 0)),
        ],
        out_specs=pl.BlockSpec((nb_g, h, b), lambda i: (i, 0, 0)),
        compiler_params=pltpu.CompilerParams(
            dimension_semantics=("parallel",)),
    )(t3, aggt3, x_t, aggx3, wu3, bu3, r3, s_t)

    return h3.transpose(2, 0, 1).astype(out_dtype)


# speedup vs baseline: 2.2595x; 1.1101x over previous
import jax
import jax.numpy as jnp
from jax.experimental import pallas as pl
from jax.experimental.pallas import tpu as pltpu

f32 = jnp.float32
bf16 = jnp.bfloat16


def _prep_a_kernel(e_ref, xf_ref, gb_ref, ub_ref,
                   a_ref, aggx_ref, bg_ref, bu_ref):
    e = e_ref[...]
    g = jax.lax.dot_general(e, e, (((1,), (1,)), ((), ())),
                            preferred_element_type=f32)
    g = jnp.maximum(g, 0.0)
    g = g - jnp.max(g, axis=1, keepdims=True)
    eg = jnp.exp(g)
    a = eg / jnp.sum(eg, axis=1, keepdims=True)
    a_ref[...] = a.astype(a_ref.dtype)
    aggx_ref[...] = jnp.dot(a, xf_ref[...], preferred_element_type=f32)
    bg_ref[...] = jnp.dot(e, gb_ref[...], preferred_element_type=f32)
    bu_ref[...] = jnp.dot(e, ub_ref[...], preferred_element_type=f32)


def _weff_kernel(e_ref, gcat_ref, ucat_ref, wg_ref, wu_ref):
    e = e_ref[...]
    wg_ref[...] = jnp.dot(e, gcat_ref[...],
                          preferred_element_type=f32).astype(wg_ref.dtype)
    wu_ref[...] = jnp.dot(e, ucat_ref[...],
                          preferred_element_type=f32).astype(wu_ref.dtype)


def _agg_kernel(a_ref, v_ref, o_ref):
    o_ref[...] = jnp.dot(a_ref[...], v_ref[...].astype(bf16),
                         preferred_element_type=f32)


def _gate_kernel(s_ref, aggs_ref, x_ref, aggx_ref, wg_ref, bg_ref,
                 t_ref, r_ref):
    nb = s_ref.shape[0]
    for j in range(nb):
        s = s_ref[j]
        cat = jnp.concatenate(
            [s, aggs_ref[j], x_ref[j], aggx_ref[j]],
            axis=0).astype(bf16)
        pre = jax.lax.dot_general(wg_ref[j], cat, (((0,), (0,)), ((), ())),
                                  preferred_element_type=f32)
        zr = jax.nn.sigmoid(pre + bg_ref[j])
        h = s.shape[0]
        z = zr[:h, :]
        t_ref[j] = z * s
        r_ref[j] = zr[h:, :]


def _cand_kernel(t_ref, aggt_ref, x_ref, aggx_ref, wu_ref, bu_ref,
                 r_ref, s_ref, h_ref):
    nb = t_ref.shape[0]
    for j in range(nb):
        cat = jnp.concatenate(
            [t_ref[j], aggt_ref[j], x_ref[j], aggx_ref[j]],
            axis=0).astype(bf16)
        pre = jax.lax.dot_general(wu_ref[j], cat, (((0,), (0,)), ((), ())),
                                  preferred_element_type=f32)
        hc = jnp.tanh(pre + bu_ref[j])
        r = r_ref[j]
        h_ref[j] = r * s_ref[j] + (1.0 - r) * hc


def kernel(x, state, node_emb, gate_w, gate_b, upd_w, upd_b):
    b, n, ci = x.shape
    h = state.shape[-1]
    d = node_emb.shape[-1]
    out_dtype = state.dtype
    kc = 2 * h + 2 * ci

    e = node_emb.astype(f32)
    x_t = x.astype(f32).transpose(1, 2, 0)
    x_flat = x_t.reshape(n, ci * b)
    s_t = state.astype(f32).transpose(1, 2, 0)
    s_flat = s_t.reshape(n, h * b)

    gw = gate_w.astype(f32)
    uw = upd_w.astype(f32)
    gcat = jnp.concatenate(
        [gw[:, 0, ci:, :], gw[:, 1, ci:, :],
         gw[:, 0, :ci, :], gw[:, 1, :ci, :]], axis=1).reshape(d, kc * 2 * h)
    ucat = jnp.concatenate(
        [uw[:, 0, ci:, :], uw[:, 1, ci:, :],
         uw[:, 0, :ci, :], uw[:, 1, :ci, :]], axis=1).reshape(d, kc * h)

    vmem = pl.BlockSpec(memory_space=pltpu.MemorySpace.VMEM)
    a_adj, aggx_flat, bg, bu = pl.pallas_call(
        _prep_a_kernel,
        out_shape=(jax.ShapeDtypeStruct((n, n), bf16),
                   jax.ShapeDtypeStruct((n, ci * b), f32),
                   jax.ShapeDtypeStruct((n, 2 * h), f32),
                   jax.ShapeDtypeStruct((n, h), f32)),
        in_specs=[vmem] * 4,
        out_specs=(vmem,) * 4,
    )(e, x_flat, gate_b.astype(f32), upd_b.astype(f32))
    bg3 = bg.reshape(n, 2 * h, 1)
    bu3 = bu.reshape(n, h, 1)

    nb_w = 64 if n % 64 == 0 else n
    wg_flat, wu_flat = pl.pallas_call(
        _weff_kernel,
        grid=(n // nb_w,),
        out_shape=(jax.ShapeDtypeStruct((n, kc * 2 * h), bf16),
                   jax.ShapeDtypeStruct((n, kc * h), bf16)),
        in_specs=[
            pl.BlockSpec((nb_w, d), lambda i: (i, 0)),
            pl.BlockSpec((d, kc * 2 * h), lambda i: (0, 0)),
            pl.BlockSpec((d, kc * h), lambda i: (0, 0)),
        ],
        out_specs=(pl.BlockSpec((nb_w, kc * 2 * h), lambda i: (i, 0)),
                   pl.BlockSpec((nb_w, kc * h), lambda i: (i, 0))),
        compiler_params=pltpu.CompilerParams(
            dimension_semantics=("parallel",)),
    )(e, gcat, ucat)
    wg3 = wg_flat.reshape(n, kc, 2 * h)
    wu3 = wu_flat.reshape(n, kc, h)

    def agg(v_flat):
        cols = v_flat.shape[1]
        cb = 2048 if cols % 2048 == 0 else cols
        return pl.pallas_call(
            _agg_kernel,
            grid=(cols // cb,),
            out_shape=jax.ShapeDtypeStruct((n, cols), f32),
            in_specs=[
                pl.BlockSpec((n, n), lambda i: (0, 0)),
                pl.BlockSpec((n, cb), lambda i: (0, i)),
            ],
            out_specs=pl.BlockSpec((n, cb), lambda i: (0, i)),
            compiler_params=pltpu.CompilerParams(
                dimension_semantics=("parallel",)),
        )(a_adj, v_flat)

    aggs3 = agg(s_flat).reshape(n, h, b)
    aggx3 = aggx_flat.reshape(n, ci, b)

    nb_g = 32 if n % 32 == 0 else n
    t3, r3 = pl.pallas_call(
        _gate_kernel,
        grid=(n // nb_g,),
        out_shape=(jax.ShapeDtypeStruct((n, h, b), f32),
                   jax.ShapeDtypeStruct((n, h, b), f32)),
        in_specs=[
            pl.BlockSpec((nb_g, h, b), lambda i: (i, 0, 0)),
            pl.BlockSpec((nb_g, h, b), lambda i: (i, 0, 0)),
            pl.BlockSpec((nb_g, ci, b), lambda i: (i, 0, 0)),
            pl.BlockSpec((nb_g, ci, b), lambda i: (i, 0, 0)),
            pl.BlockSpec((nb_g, kc, 2 * h), lambda i: (i, 0, 0)),
            pl.BlockSpec((nb_g, 2 * h, 1), lambda i: (i, 0, 0)),
        ],
        out_specs=(pl.BlockSpec((nb_g, h, b), lambda i: (i, 0, 0)),
                   pl.BlockSpec((nb_g, h, b), lambda i: (i, 0, 0))),
        compiler_params=pltpu.CompilerParams(
            dimension_semantics=("parallel",)),
    )(s_t, aggs3, x_t, aggx3, wg3, bg3)

    aggt3 = agg(t3.reshape(n, h * b)).reshape(n, h, b)

    h3 = pl.pallas_call(
        _cand_kernel,
        grid=(n // nb_g,),
        out_shape=jax.ShapeDtypeStruct((n, h, b), f32),
        in_specs=[
            pl.BlockSpec((nb_g, h, b), lambda i: (i, 0, 0)),
            pl.BlockSpec((nb_g, h, b), lambda i: (i, 0, 0)),
            pl.BlockSpec((nb_g, ci, b), lambda i: (i, 0, 0)),
            pl.BlockSpec((nb_g, ci, b), lambda i: (i, 0, 0)),
            pl.BlockSpec((nb_g, kc, h), lambda i: (i, 0, 0)),
            pl.BlockSpec((nb_g, h, 1), lambda i: (i, 0, 0)),
            pl.BlockSpec((nb_g, h, b), lambda i: (i, 0, 0)),
            pl.BlockSpec((nb_g, h, b), lambda i: (i, 0, 0)),
        ],
        out_specs=pl.BlockSpec((nb_g, h, b), lambda i: (i, 0, 0)),
        compiler_params=pltpu.CompilerParams(
            dimension_semantics=("parallel",)),
    )(t3, aggt3, x_t, aggx3, wu3, bu3, r3, s_t)

    return h3.transpose(2, 0, 1).astype(out_dtype)


# speedup vs baseline: 2.7719x; 1.2268x over previous
import jax
import jax.numpy as jnp
from jax.experimental import pallas as pl
from jax.experimental.pallas import tpu as pltpu

f32 = jnp.float32
bf16 = jnp.bfloat16


def _prep_a_kernel(e_ref, xf_ref, gb_ref, ub_ref,
                   a_ref, aggx_ref, bg_ref, bu_ref):
    e = e_ref[...]
    g = jax.lax.dot_general(e, e, (((1,), (1,)), ((), ())),
                            preferred_element_type=f32)
    g = jnp.maximum(g, 0.0)
    g = g - jnp.max(g, axis=1, keepdims=True)
    eg = jnp.exp(g)
    a = eg / jnp.sum(eg, axis=1, keepdims=True)
    a_b = a.astype(bf16)
    a_ref[...] = a_b
    aggx_ref[...] = jnp.dot(a_b, xf_ref[...],
                            preferred_element_type=f32).astype(bf16)
    bg_ref[...] = jnp.dot(e, gb_ref[...], preferred_element_type=f32)
    bu_ref[...] = jnp.dot(e, ub_ref[...], preferred_element_type=f32)


def _weff_kernel(e_ref, gcat_ref, ucat_ref, wg_ref, wu_ref):
    e = e_ref[...]
    wg_ref[...] = jnp.dot(e, gcat_ref[...],
                          preferred_element_type=f32).astype(wg_ref.dtype)
    wu_ref[...] = jnp.dot(e, ucat_ref[...],
                          preferred_element_type=f32).astype(wu_ref.dtype)


def _agg_kernel(a_ref, v_ref, o_ref):
    o_ref[...] = jnp.dot(a_ref[...], v_ref[...],
                         preferred_element_type=f32).astype(bf16)


def _gate_kernel(s_ref, aggs_ref, x_ref, aggx_ref, wg_ref, bg_ref,
                 t_ref, r_ref):
    nb = s_ref.shape[0]
    for j in range(nb):
        s = s_ref[j]
        cat = jnp.concatenate(
            [s, aggs_ref[j], x_ref[j], aggx_ref[j]], axis=0)
        pre = jax.lax.dot_general(wg_ref[j], cat, (((0,), (0,)), ((), ())),
                                  preferred_element_type=f32)
        zr = jax.nn.sigmoid(pre + bg_ref[j])
        h = s.shape[0]
        z = zr[:h, :]
        t_ref[j] = (z * s.astype(f32)).astype(bf16)
        r_ref[j] = zr[h:, :]


def _cand_kernel(t_ref, aggt_ref, x_ref, aggx_ref, wu_ref, bu_ref,
                 r_ref, s_ref, h_ref):
    nb = t_ref.shape[0]
    for j in range(nb):
        cat = jnp.concatenate(
            [t_ref[j], aggt_ref[j], x_ref[j], aggx_ref[j]], axis=0)
        pre = jax.lax.dot_general(wu_ref[j], cat, (((0,), (0,)), ((), ())),
                                  preferred_element_type=f32)
        hc = jnp.tanh(pre + bu_ref[j])
        r = r_ref[j]
        h_ref[j] = r * s_ref[j].astype(f32) + (1.0 - r) * hc


def kernel(x, state, node_emb, gate_w, gate_b, upd_w, upd_b):
    b, n, ci = x.shape
    h = state.shape[-1]
    d = node_emb.shape[-1]
    out_dtype = state.dtype
    kc = 2 * h + 2 * ci

    e = node_emb.astype(f32)
    x_t = x.astype(bf16).transpose(1, 2, 0)
    x_flat = x_t.reshape(n, ci * b)
    s_t = state.astype(bf16).transpose(1, 2, 0)
    s_flat = s_t.reshape(n, h * b)

    gw = gate_w.astype(f32)
    uw = upd_w.astype(f32)
    gcat = jnp.concatenate(
        [gw[:, 0, ci:, :], gw[:, 1, ci:, :],
         gw[:, 0, :ci, :], gw[:, 1, :ci, :]], axis=1).reshape(d, kc * 2 * h)
    ucat = jnp.concatenate(
        [uw[:, 0, ci:, :], uw[:, 1, ci:, :],
         uw[:, 0, :ci, :], uw[:, 1, :ci, :]], axis=1).reshape(d, kc * h)

    vmem = pl.BlockSpec(memory_space=pltpu.MemorySpace.VMEM)
    a_adj, aggx_flat, bg, bu = pl.pallas_call(
        _prep_a_kernel,
        out_shape=(jax.ShapeDtypeStruct((n, n), bf16),
                   jax.ShapeDtypeStruct((n, ci * b), bf16),
                   jax.ShapeDtypeStruct((n, 2 * h), f32),
                   jax.ShapeDtypeStruct((n, h), f32)),
        in_specs=[vmem] * 4,
        out_specs=(vmem,) * 4,
    )(e, x_flat, gate_b.astype(f32), upd_b.astype(f32))
    bg3 = bg.reshape(n, 2 * h, 1)
    bu3 = bu.reshape(n, h, 1)

    nb_w = 64 if n % 64 == 0 else n
    wg_flat, wu_flat = pl.pallas_call(
        _weff_kernel,
        grid=(n // nb_w,),
        out_shape=(jax.ShapeDtypeStruct((n, kc * 2 * h), bf16),
                   jax.ShapeDtypeStruct((n, kc * h), bf16)),
        in_specs=[
            pl.BlockSpec((nb_w, d), lambda i: (i, 0)),
            pl.BlockSpec((d, kc * 2 * h), lambda i: (0, 0)),
            pl.BlockSpec((d, kc * h), lambda i: (0, 0)),
        ],
        out_specs=(pl.BlockSpec((nb_w, kc * 2 * h), lambda i: (i, 0)),
                   pl.BlockSpec((nb_w, kc * h), lambda i: (i, 0))),
        compiler_params=pltpu.CompilerParams(
            dimension_semantics=("parallel",)),
    )(e, gcat, ucat)
    wg3 = wg_flat.reshape(n, kc, 2 * h)
    wu3 = wu_flat.reshape(n, kc, h)

    def agg(v_flat):
        cols = v_flat.shape[1]
        cb = 2048 if cols % 2048 == 0 else cols
        return pl.pallas_call(
            _agg_kernel,
            grid=(cols // cb,),
            out_shape=jax.ShapeDtypeStruct((n, cols), bf16),
            in_specs=[
                pl.BlockSpec((n, n), lambda i: (0, 0)),
                pl.BlockSpec((n, cb), lambda i: (0, i)),
            ],
            out_specs=pl.BlockSpec((n, cb), lambda i: (0, i)),
            compiler_params=pltpu.CompilerParams(
                dimension_semantics=("parallel",)),
        )(a_adj, v_flat)

    aggs3 = agg(s_flat).reshape(n, h, b)
    aggx3 = aggx_flat.reshape(n, ci, b)

    nb_g = 32 if n % 32 == 0 else n
    t3, r3 = pl.pallas_call(
        _gate_kernel,
        grid=(n // nb_g,),
        out_shape=(jax.ShapeDtypeStruct((n, h, b), bf16),
                   jax.ShapeDtypeStruct((n, h, b), f32)),
        in_specs=[
            pl.BlockSpec((nb_g, h, b), lambda i: (i, 0, 0)),
            pl.BlockSpec((nb_g, h, b), lambda i: (i, 0, 0)),
            pl.BlockSpec((nb_g, ci, b), lambda i: (i, 0, 0)),
            pl.BlockSpec((nb_g, ci, b), lambda i: (i, 0, 0)),
            pl.BlockSpec((nb_g, kc, 2 * h), lambda i: (i, 0, 0)),
            pl.BlockSpec((nb_g, 2 * h, 1), lambda i: (i, 0, 0)),
        ],
        out_specs=(pl.BlockSpec((nb_g, h, b), lambda i: (i, 0, 0)),
                   pl.BlockSpec((nb_g, h, b), lambda i: (i, 0, 0))),
        compiler_params=pltpu.CompilerParams(
            dimension_semantics=("parallel",)),
    )(s_t, aggs3, x_t, aggx3, wg3, bg3)

    aggt3 = agg(t3.reshape(n, h * b)).reshape(n, h, b)

    h3 = pl.pallas_call(
        _cand_kernel,
        grid=(n // nb_g,),
        out_shape=jax.ShapeDtypeStruct((n, h, b), f32),
        in_specs=[
            pl.BlockSpec((nb_g, h, b), lambda i: (i, 0, 0)),
            pl.BlockSpec((nb_g, h, b), lambda i: (i, 0, 0)),
            pl.BlockSpec((nb_g, ci, b), lambda i: (i, 0, 0)),
            pl.BlockSpec((nb_g, ci, b), lambda i: (i, 0, 0)),
            pl.BlockSpec((nb_g, kc, h), lambda i: (i, 0, 0)),
            pl.BlockSpec((nb_g, h, 1), lambda i: (i, 0, 0)),
            pl.BlockSpec((nb_g, h, b), lambda i: (i, 0, 0)),
            pl.BlockSpec((nb_g, h, b), lambda i: (i, 0, 0)),
        ],
        out_specs=pl.BlockSpec((nb_g, h, b), lambda i: (i, 0, 0)),
        compiler_params=pltpu.CompilerParams(
            dimension_semantics=("parallel",)),
    )(t3, aggt3, x_t, aggx3, wu3, bu3, r3, s_t)

    return h3.transpose(2, 0, 1).astype(out_dtype)


# speedup vs baseline: 3.8618x; 1.3932x over previous
import jax
import jax.numpy as jnp
from jax.experimental import pallas as pl
from jax.experimental.pallas import tpu as pltpu

f32 = jnp.float32
bf16 = jnp.bfloat16


def _prep_a_kernel(e_ref, x_ref, gb_ref, ub_ref,
                   a_ref, aggx_ref, bg_ref, bu_ref):
    e = e_ref[...]
    g = jax.lax.dot_general(e, e, (((1,), (1,)), ((), ())),
                            preferred_element_type=f32)
    g = jnp.maximum(g, 0.0)
    g = g - jnp.max(g, axis=1, keepdims=True)
    eg = jnp.exp(g)
    a = eg / jnp.sum(eg, axis=1, keepdims=True)
    a_b = a.astype(bf16)
    a_ref[...] = a_b
    nn, ci, bb = x_ref.shape
    x2 = x_ref[...].reshape(nn, ci * bb)
    aggx = jnp.dot(a_b, x2, preferred_element_type=f32).astype(bf16)
    aggx_ref[...] = aggx.reshape(nn, ci, bb)
    bg = jnp.dot(e, gb_ref[...], preferred_element_type=f32)
    bu = jnp.dot(e, ub_ref[...], preferred_element_type=f32)
    bg_ref[...] = bg.reshape(bg.shape[0], bg.shape[1], 1)
    bu_ref[...] = bu.reshape(bu.shape[0], bu.shape[1], 1)


def _weff_kernel(e_ref, gcat_ref, ucat_ref, wg_ref, wu_ref):
    e = e_ref[...]
    nb, kc, og = wg_ref.shape
    ou = wu_ref.shape[2]
    wg = jnp.dot(e, gcat_ref[...], preferred_element_type=f32).astype(bf16)
    wu = jnp.dot(e, ucat_ref[...], preferred_element_type=f32).astype(bf16)
    wg_ref[...] = wg.reshape(nb, kc, og)
    wu_ref[...] = wu.reshape(nb, kc, ou)


def _agg_kernel(a_ref, v_ref, o_ref):
    nn, hb, bb = v_ref.shape
    v2 = v_ref[...].reshape(nn, hb * bb)
    o2 = jnp.dot(a_ref[...], v2, preferred_element_type=f32).astype(bf16)
    o_ref[...] = o2.reshape(nn, hb, bb)


def _gate_kernel(s_ref, aggs_ref, x_ref, aggx_ref, wg_ref, bg_ref,
                 t_ref, r_ref):
    nb = s_ref.shape[0]
    for j in range(nb):
        s = s_ref[j]
        cat = jnp.concatenate(
            [s, aggs_ref[j], x_ref[j], aggx_ref[j]], axis=0)
        pre = jax.lax.dot_general(wg_ref[j], cat, (((0,), (0,)), ((), ())),
                                  preferred_element_type=f32)
        zr = jax.nn.sigmoid(pre + bg_ref[j])
        h = s.shape[0]
        z = zr[:h, :]
        t_ref[j] = (z * s.astype(f32)).astype(bf16)
        r_ref[j] = zr[h:, :]


def _cand_kernel(t_ref, aggt_ref, x_ref, aggx_ref, wu_ref, bu_ref,
                 r_ref, s_ref, h_ref):
    nb = t_ref.shape[0]
    for j in range(nb):
        cat = jnp.concatenate(
            [t_ref[j], aggt_ref[j], x_ref[j], aggx_ref[j]], axis=0)
        pre = jax.lax.dot_general(wu_ref[j], cat, (((0,), (0,)), ((), ())),
                                  preferred_element_type=f32)
        hc = jnp.tanh(pre + bu_ref[j])
        r = r_ref[j]
        h_ref[j] = r * s_ref[j].astype(f32) + (1.0 - r) * hc


def kernel(x, state, node_emb, gate_w, gate_b, upd_w, upd_b):
    b, n, ci = x.shape
    h = state.shape[-1]
    d = node_emb.shape[-1]
    out_dtype = state.dtype
    kc = 2 * h + 2 * ci

    e = node_emb.astype(f32)
    x_t = x.astype(bf16).transpose(1, 2, 0)
    s_t = state.astype(bf16).transpose(1, 2, 0)

    gw = gate_w.astype(f32)
    uw = upd_w.astype(f32)
    gcat = jnp.concatenate(
        [gw[:, 0, ci:, :], gw[:, 1, ci:, :],
         gw[:, 0, :ci, :], gw[:, 1, :ci, :]], axis=1).reshape(d, kc * 2 * h)
    ucat = jnp.concatenate(
        [uw[:, 0, ci:, :], uw[:, 1, ci:, :],
         uw[:, 0, :ci, :], uw[:, 1, :ci, :]], axis=1).reshape(d, kc * h)

    vmem = pl.BlockSpec(memory_space=pltpu.MemorySpace.VMEM)
    a_adj, aggx3, bg3, bu3 = pl.pallas_call(
        _prep_a_kernel,
        out_shape=(jax.ShapeDtypeStruct((n, n), bf16),
                   jax.ShapeDtypeStruct((n, ci, b), bf16),
                   jax.ShapeDtypeStruct((n, 2 * h, 1), f32),
                   jax.ShapeDtypeStruct((n, h, 1), f32)),
        in_specs=[vmem] * 4,
        out_specs=(vmem,) * 4,
    )(e, x_t, gate_b.astype(f32), upd_b.astype(f32))

    nb_w = 64 if n % 64 == 0 else n
    wg3, wu3 = pl.pallas_call(
        _weff_kernel,
        grid=(n // nb_w,),
        out_shape=(jax.ShapeDtypeStruct((n, kc, 2 * h), bf16),
                   jax.ShapeDtypeStruct((n, kc, h), bf16)),
        in_specs=[
            pl.BlockSpec((nb_w, d), lambda i: (i, 0)),
            pl.BlockSpec((d, kc * 2 * h), lambda i: (0, 0)),
            pl.BlockSpec((d, kc * h), lambda i: (0, 0)),
        ],
        out_specs=(pl.BlockSpec((nb_w, kc, 2 * h), lambda i: (i, 0, 0)),
                   pl.BlockSpec((nb_w, kc, h), lambda i: (i, 0, 0))),
        compiler_params=pltpu.CompilerParams(
            dimension_semantics=("parallel",)),
    )(e, gcat, ucat)

    def agg(v3):
        feat = v3.shape[1]
        fb = 8 if feat % 8 == 0 else feat
        return pl.pallas_call(
            _agg_kernel,
            grid=(feat // fb,),
            out_shape=jax.ShapeDtypeStruct((n, feat, b), bf16),
            in_specs=[
                pl.BlockSpec((n, n), lambda i: (0, 0)),
                pl.BlockSpec((n, fb, b), lambda i: (0, i, 0)),
            ],
            out_specs=pl.BlockSpec((n, fb, b), lambda i: (0, i, 0)),
            compiler_params=pltpu.CompilerParams(
                dimension_semantics=("parallel",)),
        )(a_adj, v3)

    aggs3 = agg(s_t)

    nb_g = 32 if n % 32 == 0 else n
    t3, r3 = pl.pallas_call(
        _gate_kernel,
        grid=(n // nb_g,),
        out_shape=(jax.ShapeDtypeStruct((n, h, b), bf16),
                   jax.ShapeDtypeStruct((n, h, b), f32)),
        in_specs=[
            pl.BlockSpec((nb_g, h, b), lambda i: (i, 0, 0)),
            pl.BlockSpec((nb_g, h, b), lambda i: (i, 0, 0)),
            pl.BlockSpec((nb_g, ci, b), lambda i: (i, 0, 0)),
            pl.BlockSpec((nb_g, ci, b), lambda i: (i, 0, 0)),
            pl.BlockSpec((nb_g, kc, 2 * h), lambda i: (i, 0, 0)),
            pl.BlockSpec((nb_g, 2 * h, 1), lambda i: (i, 0, 0)),
        ],
        out_specs=(pl.BlockSpec((nb_g, h, b), lambda i: (i, 0, 0)),
                   pl.BlockSpec((nb_g, h, b), lambda i: (i, 0, 0))),
        compiler_params=pltpu.CompilerParams(
            dimension_semantics=("parallel",)),
    )(s_t, aggs3, x_t, aggx3, wg3, bg3)

    aggt3 = agg(t3)

    h3 = pl.pallas_call(
        _cand_kernel,
        grid=(n // nb_g,),
        out_shape=jax.ShapeDtypeStruct((n, h, b), f32),
        in_specs=[
            pl.BlockSpec((nb_g, h, b), lambda i: (i, 0, 0)),
            pl.BlockSpec((nb_g, h, b), lambda i: (i, 0, 0)),
            pl.BlockSpec((nb_g, ci, b), lambda i: (i, 0, 0)),
            pl.BlockSpec((nb_g, ci, b), lambda i: (i, 0, 0)),
            pl.BlockSpec((nb_g, kc, h), lambda i: (i, 0, 0)),
            pl.BlockSpec((nb_g, h, 1), lambda i: (i, 0, 0)),
            pl.BlockSpec((nb_g, h, b), lambda i: (i, 0, 0)),
            pl.BlockSpec((nb_g, h, b), lambda i: (i, 0, 0)),
        ],
        out_specs=pl.BlockSpec((nb_g, h, b), lambda i: (i, 0, 0)),
        compiler_params=pltpu.CompilerParams(
            dimension_semantics=("parallel",)),
    )(t3, aggt3, x_t, aggx3, wu3, bu3, r3, s_t)

    return h3.transpose(2, 0, 1).astype(out_dtype)
